# baseline (speedup 1.0000x reference)
import functools

import jax
import jax.numpy as jnp
from jax import lax
from jax.experimental import pallas as pl
from jax.experimental.pallas import tpu as pltpu
from jax.experimental.pallas import tpu_sc as plsc

NC = 2
NS = 16
NW = NC * NS
NBUF = 8
BCH = 2
ROWW = 128

_MESH = plsc.VectorSubcoreMesh(
    core_axis_name="c", subcore_axis_name="s", num_cores=NC, num_subcores=NS
)


@functools.lru_cache(maxsize=None)
def _make_permute(batch, hist):
    b_per_w = batch // NW
    hpad = ROWW // BCH
    assert hist <= hpad < 2 * hist and hpad % 8 == 0
    n_ch = b_per_w // BCH
    n_grp = ROWW // 16

    @functools.partial(
        pl.kernel,
        out_type=jax.ShapeDtypeStruct((NW * n_ch, ROWW), jnp.int32),
        mesh=_MESH,
        scratch_types=[
            pltpu.VMEM((hist, b_per_w), jnp.int32),
            pltpu.VMEM((n_ch, ROWW), jnp.int32),
        ],
        compiler_params=pltpu.CompilerParams(needs_layout_passes=False),
    )
    def permute_kernel(in_hbm, out_hbm, buf_in, buf_out):
        wid = lax.axis_index("s") * NC + lax.axis_index("c")
        b0 = wid * b_per_w
        pltpu.sync_copy(in_hbm.at[:, pl.ds(b0, b_per_w)], buf_in)

        lanes = lax.iota(jnp.int32, 16)
        h_pat, boff_pat = [], []
        for g in range(n_grp):
            j = lanes + (g * 16)
            boff = jnp.zeros((16,), jnp.int32)
            for _ in range(BCH):
                wrap = j >= hpad
                j = jnp.where(wrap, j - hpad, j)
                boff = boff + wrap.astype(jnp.int32)
            h_pat.append(jnp.where(j >= hist, j - hist, j))
            boff_pat.append(boff)

        @pl.loop(0, n_ch, init_carry=jnp.zeros((16,), jnp.int32))
        def _chunk(cc, c):
            for g in range(n_grp):
                vals = plsc.load_gather(buf_in, [h_pat[g], c + boff_pat[g]])
                buf_out[cc, pl.ds(g * 16, 16)] = vals
            return c + BCH

        pltpu.sync_copy(buf_out, out_hbm.at[pl.ds(wid * n_ch, n_ch)])

    return permute_kernel


@functools.lru_cache(maxsize=None)
def _make_gather(batch, hist, d):
    assert batch % NW == 0
    b_per_w = batch // NW
    n_ch = b_per_w // BCH
    assert b_per_w % BCH == 0 and n_ch % NBUF == 0
    hpad = ROWW // BCH

    @functools.partial(
        pl.kernel,
        out_type=jax.ShapeDtypeStruct((batch, hist, d), jnp.float32),
        mesh=_MESH,
        scratch_types=[
            pltpu.VMEM((n_ch, ROWW), jnp.int32),
            pltpu.VMEM((NBUF, BCH, hpad, d), jnp.float32),
        ]
        + [pltpu.SemaphoreType.DMA] * (2 * NBUF),
        compiler_params=pltpu.CompilerParams(
            use_tc_tiling_on_sc=False, needs_layout_passes=False
        ),
    )
    def gather_kernel(table_hbm, idx_hbm, out_hbm, idx_v, ring, *sems):
        gsems = sems[:NBUF]
        wsems = sems[NBUF:]
        wid = lax.axis_index("s") * NC + lax.axis_index("c")
        b0 = wid * b_per_w
        pltpu.sync_copy(idx_hbm.at[pl.ds(wid * n_ch, n_ch)], idx_v)

        def start_gather(slot, cc):
            for b2 in range(BCH):
                pltpu.async_copy(
                    table_hbm.at[idx_v.at[cc, pl.ds(b2 * hpad, hpad)]],
                    ring.at[slot, b2],
                    gsems[slot],
                )

        def wait_gather(slot):
            for b2 in range(BCH):
                pltpu.make_async_copy(
                    table_hbm.at[pl.ds(0, hpad)],
                    ring.at[slot, b2],
                    gsems[slot],
                ).wait()

        def start_write(slot, cc):
            pltpu.async_copy(
                ring.at[slot, :, pl.ds(0, hist)],
                out_hbm.at[pl.ds(b0 + cc * BCH, BCH)],
                wsems[slot],
            )

        def wait_write(slot):
            pltpu.make_async_copy(
                ring.at[slot, :, pl.ds(0, hist)],
                out_hbm.at[pl.ds(0, BCH)],
                wsems[slot],
            ).wait()

        for s in range(NBUF):
            start_gather(s, s)

        @pl.loop(0, n_ch - NBUF, step=NBUF)
        def _outer(g):
            for s in range(NBUF):
                wait_gather(s)
                start_write(s, g + s)
            for s in range(NBUF):
                wait_write(s)
                start_gather(s, g + s + NBUF)

        g0 = n_ch - NBUF
        for s in range(NBUF):
            wait_gather(s)
            start_write(s, g0 + s)
        for s in range(NBUF):
            wait_write(s)

    return gather_kernel


def kernel(embedding_matrix, inputs):
    b, h = inputs.shape
    d = embedding_matrix.shape[1]
    inputs_t = inputs.T.astype(jnp.int32)
    idx_rows = _make_permute(b, h)(inputs_t)
    return _make_gather(b, h, d)(embedding_matrix, idx_rows)
